# Initial kernel scaffold; baseline (speedup 1.0000x reference)
import jax
import jax.numpy as jnp
from jax.experimental import pallas as pl


def kernel(x, table):
    raise NotImplementedError("write your pallas kernel here")


# speedup vs baseline: 2.1908x; 2.1908x over previous
import functools

import jax
import jax.numpy as jnp
from jax import lax
from jax.experimental import pallas as pl
from jax.experimental.pallas import tpu as pltpu
from jax.experimental.pallas import tpu_sc as plsc

VOCAB = 256
EMBED_DIM = 100
D2 = 2 * EMBED_DIM

NC = 2
NS = 16
NW = NC * NS

G = 128
JG = 2
CHUNK = G * JG


def _emb_kernel(n_chunks_per_w):
    def body(idx_hbm, table_hbm, out_hbm, idx_v, rows_v, sem):
        wid = lax.axis_index("s") * NC + lax.axis_index("c")

        def chunk_body(g, carry):
            c = wid * n_chunks_per_w + g
            pltpu.sync_copy(idx_hbm.at[c], idx_v)
            copies = []
            for j in range(JG):
                copies.append(
                    pltpu.async_copy(
                        table_hbm.at[idx_v.at[j]],
                        rows_v.at[pl.ds(j * G, G)],
                        sem,
                    )
                )
            for cp in copies:
                cp.wait()
            pltpu.sync_copy(rows_v, out_hbm.at[pl.ds(c * CHUNK, CHUNK)])
            return carry

        lax.fori_loop(0, n_chunks_per_w, chunk_body, 0)

    return body


def kernel(x, table):
    B, S = x.shape
    n = B * S
    npairs = n // 2
    assert npairs % (NW * CHUNK) == 0
    n_chunks_per_w = npairs // (NW * CHUNK)

    x2 = x.reshape(npairs, 2).astype(jnp.int32)
    idx2 = (x2[:, 0] * VOCAB + x2[:, 1]).reshape(npairs // CHUNK, JG, G)

    table2 = jnp.concatenate(
        [
            jnp.broadcast_to(table[:, None, :], (VOCAB, VOCAB, EMBED_DIM)),
            jnp.broadcast_to(table[None, :, :], (VOCAB, VOCAB, EMBED_DIM)),
        ],
        axis=-1,
    ).reshape(VOCAB * VOCAB, D2)

    mesh = plsc.VectorSubcoreMesh(core_axis_name="c", subcore_axis_name="s")
    run = functools.partial(
        pl.kernel,
        mesh=mesh,
        out_type=jax.ShapeDtypeStruct((npairs, D2), jnp.float32),
        scratch_types=[
            pltpu.VMEM((JG, G), jnp.int32),
            pltpu.VMEM((CHUNK, D2), jnp.float32),
            pltpu.SemaphoreType.DMA,
        ],
        compiler_params=pltpu.CompilerParams(use_tc_tiling_on_sc=False),
    )(_emb_kernel(n_chunks_per_w))

    out = run(idx2, table2)
    return out.reshape(B, S, EMBED_DIM)


# speedup vs baseline: 2.2623x; 1.0327x over previous
import functools

import jax
import jax.numpy as jnp
from jax import lax
from jax.experimental import pallas as pl
from jax.experimental.pallas import tpu as pltpu
from jax.experimental.pallas import tpu_sc as plsc

VOCAB = 256
EMBED_DIM = 100
D2 = 2 * EMBED_DIM

NC = 2
NS = 16
NW = NC * NS

G = 128
JG = 2
CHUNK = G * JG
NBUF = 2


def _emb_kernel(n_chunks_per_w):
    n_iter = n_chunks_per_w // NBUF

    def body(idx_hbm, table_hbm, out_hbm, idx_v, rows_v, idx_sem, gat_sem,
             out_sem):
        wid = lax.axis_index("s") * NC + lax.axis_index("c")
        base = wid * n_chunks_per_w

        def drain_idx(b):
            pltpu.make_async_copy(idx_hbm.at[0], idx_v.at[b],
                                  idx_sem.at[b]).wait()

        def drain_gat(b):
            pltpu.make_async_copy(out_hbm.at[pl.ds(0, CHUNK)], rows_v.at[b],
                                  gat_sem.at[b]).wait()

        def drain_out(b):
            pltpu.make_async_copy(rows_v.at[b], out_hbm.at[pl.ds(0, CHUNK)],
                                  out_sem.at[b]).wait()

        pltpu.sync_copy(idx_hbm.at[base], idx_v.at[0])

        def loop_body(t, carry):
            for b in range(NBUF):
                g = t * NBUF + b
                if b == 0:
                    @pl.when(t > 0)
                    def _():
                        drain_idx(b)
                        drain_out(b)
                else:
                    drain_idx(b)

                    @pl.when(t > 0)
                    def _():
                        drain_out(b)

                for j in range(JG):
                    pltpu.async_copy(
                        table_hbm.at[idx_v.at[b].at[j]],
                        rows_v.at[b].at[pl.ds(j * G, G)],
                        gat_sem.at[b],
                    )
                bn = (b + 1) % NBUF
                if b < NBUF - 1:
                    pltpu.async_copy(idx_hbm.at[base + g + 1], idx_v.at[bn],
                                     idx_sem.at[bn])
                else:
                    @pl.when(t < n_iter - 1)
                    def _():
                        pltpu.async_copy(idx_hbm.at[base + g + 1],
                                         idx_v.at[bn], idx_sem.at[bn])

                drain_gat(b)
                pltpu.async_copy(rows_v.at[b],
                                 out_hbm.at[pl.ds((base + g) * CHUNK, CHUNK)],
                                 out_sem.at[b])
            return carry

        lax.fori_loop(0, n_iter, loop_body, 0)
        for b in range(NBUF):
            drain_out(b)

    return body


def kernel(x, table):
    B, S = x.shape
    n = B * S
    npairs = n // 2
    assert npairs % (NW * CHUNK * NBUF) == 0
    n_chunks_per_w = npairs // (NW * CHUNK)

    x2 = x.reshape(npairs, 2).astype(jnp.int32)
    idx2 = (x2[:, 0] * VOCAB + x2[:, 1]).reshape(npairs // CHUNK, JG, G)

    table2 = jnp.concatenate(
        [
            jnp.broadcast_to(table[:, None, :], (VOCAB, VOCAB, EMBED_DIM)),
            jnp.broadcast_to(table[None, :, :], (VOCAB, VOCAB, EMBED_DIM)),
        ],
        axis=-1,
    ).reshape(VOCAB * VOCAB, D2)

    mesh = plsc.VectorSubcoreMesh(core_axis_name="c", subcore_axis_name="s")
    run = functools.partial(
        pl.kernel,
        mesh=mesh,
        out_type=jax.ShapeDtypeStruct((npairs, D2), jnp.float32),
        scratch_types=[
            pltpu.VMEM((NBUF, JG, G), jnp.int32),
            pltpu.VMEM((NBUF, CHUNK, D2), jnp.float32),
            pltpu.SemaphoreType.DMA((NBUF,)),
            pltpu.SemaphoreType.DMA((NBUF,)),
            pltpu.SemaphoreType.DMA((NBUF,)),
        ],
        compiler_params=pltpu.CompilerParams(use_tc_tiling_on_sc=False),
    )(_emb_kernel(n_chunks_per_w))

    out = run(idx2, table2)
    return out.reshape(B, S, EMBED_DIM)
